# baseline (device time: 134667 ns/iter reference)
import jax
import jax.numpy as jnp
from jax import lax
from jax.experimental import pallas as pl
from jax.experimental.pallas import tpu as pltpu

N_DEV = 32
ZS = 4
YS = 4
GB = 8


def kernel(x):
    _, m, n_total = x.shape
    n_per = n_total // N_DEV
    xr = x.reshape(m, ZS, YS, 2 * n_per)

    def body(x_ref, out_ref, pa_ref, recva, recvb,
             sa_sems, ra_sems, sb_sems, rb_sems):
        my = lax.axis_index("i")
        z = my // GB
        w = my % GB
        y = w // 2
        xbit = ((w + 1) // 2) % 2

        rdmas_a = []
        for k in range(1, YS):
            y_t = lax.rem(y + k, YS)
            w_t = 2 * y_t + lax.rem(xbit + y_t, 2)
            rdma = pltpu.make_async_remote_copy(
                src_ref=x_ref.at[:, :, y_t, :],
                dst_ref=recva.at[k - 1],
                send_sem=sa_sems.at[k - 1],
                recv_sem=ra_sems.at[k - 1],
                device_id=(GB * z + w_t,),
                device_id_type=pl.DeviceIdType.MESH,
            )
            rdma.start()
            rdmas_a.append(rdma)

        pa = x_ref[:, :, y, :]
        for k in range(1, YS):
            rdmas_a[k - 1].wait_recv()
            pa = pa + recva[k - 1]
        pa_ref[:, :, :] = pa

        u = ZS * xbit + z
        rdmas_b = []
        for o in range(1, GB):
            u_t = lax.rem(u + o, GB)
            x_t = u_t // ZS
            z_t = lax.rem(u_t, ZS)
            b_t = lax.rem(x_t + y, 2)
            rdma = pltpu.make_async_remote_copy(
                src_ref=pa_ref.at[:, z_t, pl.ds(b_t * n_per, n_per)],
                dst_ref=recvb.at[o - 1],
                send_sem=sb_sems.at[o - 1],
                recv_sem=rb_sems.at[o - 1],
                device_id=(GB * z_t + 2 * y + b_t,),
                device_id_type=pl.DeviceIdType.MESH,
            )
            rdma.start()
            rdmas_b.append(rdma)

        bb = lax.rem(xbit + y, 2)
        acc = pa_ref[:, z, pl.ds(bb * n_per, n_per)]
        for o in range(1, GB):
            rdmas_b[o - 1].wait_recv()
            acc = acc + recvb[o - 1]
        out_ref[:, :] = acc

        for r in rdmas_a + rdmas_b:
            r.wait_send()

    return pl.pallas_call(
        body,
        out_shape=jax.ShapeDtypeStruct((m, n_per), x.dtype),
        in_specs=[pl.BlockSpec(memory_space=pltpu.VMEM)],
        out_specs=pl.BlockSpec(memory_space=pltpu.VMEM),
        scratch_shapes=[
            pltpu.VMEM((m, ZS, 2 * n_per), x.dtype),
            pltpu.VMEM((YS - 1, m, ZS, 2 * n_per), x.dtype),
            pltpu.VMEM((GB - 1, m, n_per), x.dtype),
            pltpu.SemaphoreType.DMA((YS - 1,)),
            pltpu.SemaphoreType.DMA((YS - 1,)),
            pltpu.SemaphoreType.DMA((GB - 1,)),
            pltpu.SemaphoreType.DMA((GB - 1,)),
        ],
    )(xr)
